# Initial kernel scaffold; baseline (speedup 1.0000x reference)
import jax
import jax.numpy as jnp
from jax.experimental import pallas as pl


def kernel(x, edge_index, batch, W1, b1, W2, b2, W3, b3):
    raise NotImplementedError("write your pallas kernel here")


# speedup vs baseline: 1.0245x; 1.0245x over previous
import functools

import jax
import jax.numpy as jnp
from jax import lax
from jax.experimental import pallas as pl
from jax.experimental.pallas import tpu as pltpu
from jax.experimental.pallas import tpu_sc as plsc

_NC = 2
_NS = 16
_L = 16
_B = 16


def _build_counts(edge_flat, n, e):
    nw = _NC * _NS
    rch = 16
    nchunks = n // rch
    cpw = nchunks // nw
    eb = 2048
    stages = e // eb
    vecs = eb // _L

    mesh = plsc.VectorSubcoreMesh(core_axis_name="c", subcore_axis_name="s")

    @functools.partial(
        pl.kernel,
        out_type=[
            jax.ShapeDtypeStruct((n * n,), jnp.float32),
            jax.ShapeDtypeStruct((n * n,), jnp.float32),
        ],
        mesh=mesh,
        compiler_params=pltpu.CompilerParams(needs_layout_passes=False),
        scratch_types=[
            pltpu.VMEM((eb,), jnp.int32),
            pltpu.VMEM((eb,), jnp.int32),
            pltpu.VMEM((rch * n,), jnp.float32),
            pltpu.VMEM((rch * n,), jnp.float32),
        ],
    )
    def scatter_kernel(edges_hbm, c_hbm, ct_hbm, rows_v, cols_v, accc_v, acct_v):
        wid = lax.axis_index("s") * _NC + lax.axis_index("c")
        ones = jnp.ones((_L,), jnp.float32)
        zeros = jnp.zeros((_L,), jnp.float32)

        def per_chunk(k_, carry):
            chunk = wid * cpw + k_
            r0 = chunk * rch

            def zero_body(i, c):
                accc_v[pl.ds(i * _L, _L)] = zeros
                acct_v[pl.ds(i * _L, _L)] = zeros
                return c

            lax.fori_loop(0, rch * n // _L, zero_body, 0)

            def stage_body(s_, c):
                pltpu.sync_copy(edges_hbm.at[pl.ds(s_ * eb, eb)], rows_v)
                pltpu.sync_copy(edges_hbm.at[pl.ds(e + s_ * eb, eb)], cols_v)

                def vec_body(j, c2):
                    r16 = rows_v[pl.ds(j * _L, _L)]
                    c16 = cols_v[pl.ds(j * _L, _L)]
                    mc = (r16 >= r0) & (r16 < r0 + rch)
                    offc = jnp.where(mc, (r16 - r0) * n + c16, 0)
                    plsc.addupdate_scatter(accc_v, [offc], ones, mask=mc)
                    mt = (c16 >= r0) & (c16 < r0 + rch)
                    offt = jnp.where(mt, (c16 - r0) * n + r16, 0)
                    plsc.addupdate_scatter(acct_v, [offt], ones, mask=mt)
                    return c2

                lax.fori_loop(0, vecs, vec_body, 0)
                return c

            lax.fori_loop(0, stages, stage_body, 0)

            pltpu.sync_copy(accc_v, c_hbm.at[pl.ds(chunk * rch * n, rch * n)])
            pltpu.sync_copy(acct_v, ct_hbm.at[pl.ds(chunk * rch * n, rch * n)])
            return carry

        lax.fori_loop(0, cpw, per_chunk, 0)

    return scatter_kernel(edge_flat)


def _lap(c2, n):
    bm = 256

    def body(c_ref, lap_ref):
        c = c_ref[...]
        adj = jnp.minimum(c, 1.0)
        deg = jnp.sum(adj, axis=1)
        i = pl.program_id(0)
        rows = lax.broadcasted_iota(jnp.int32, (bm, n), 0) + i * bm
        cols = lax.broadcasted_iota(jnp.int32, (bm, n), 1)
        lap_ref[...] = jnp.where(rows == cols, deg[:, None], 0.0) - adj

    return pl.pallas_call(
        body,
        grid=(n // bm,),
        in_specs=[pl.BlockSpec((bm, n), lambda i: (i, 0))],
        out_specs=pl.BlockSpec((bm, n), lambda i: (i, 0)),
        out_shape=jax.ShapeDtypeStruct((n, n), jnp.float32),
    )(c2)


def _post_pool(h4, batch2):
    n, d = h4.shape

    def body(h_ref, bt_ref, o_ref):
        bid = lax.broadcasted_iota(jnp.int32, (_B, n), 0)
        m = (bt_ref[...] == bid).astype(jnp.float32)
        pooled = jnp.dot(m, h_ref[...], preferred_element_type=jnp.float32)
        counts = jnp.sum(m, axis=1)[:, None]
        o_ref[...] = pooled / jnp.maximum(counts, 1.0)

    return pl.pallas_call(
        body,
        out_shape=jax.ShapeDtypeStruct((_B, d), jnp.float32),
    )(h4, batch2)


def _layers(ct2, h0, w1, b1, w2, b2, w3, b3):
    n = ct2.shape[0]
    dout = w3.shape[0]

    def body(ct_ref, h_ref, w1_ref, b1_ref, w2_ref, b2_ref, w3_ref, b3_ref,
             o_ref):
        ct = ct_ref[...]
        deg = jnp.sum(ct, axis=1) + 1.0
        dinv = lax.rsqrt(deg)[:, None]

        def layer(h, w, b, relu):
            xw = lax.dot_general(h, w, (((1,), (1,)), ((), ())),
                                 preferred_element_type=jnp.float32)
            y = dinv * xw
            s = jnp.dot(ct, y, preferred_element_type=jnp.float32) + y
            r = dinv * s + b
            return jnp.maximum(r, 0.0) if relu else r

        h = layer(h_ref[...], w1_ref[...], b1_ref[...], True)
        h = layer(h, w2_ref[...], b2_ref[...], True)
        o_ref[...] = layer(h, w3_ref[...], b3_ref[...], False)

    return pl.pallas_call(
        body,
        out_shape=jax.ShapeDtypeStruct((n, dout), jnp.float32),
    )(ct2, h0, w1, b1.reshape(1, -1), w2, b2.reshape(1, -1),
      w3, b3.reshape(1, -1))


def kernel(x, edge_index, batch, W1, b1, W2, b2, W3, b3):
    n = x.shape[0]
    e = edge_index.shape[1]
    cflat, ctflat = _build_counts(edge_index.reshape(-1), n, e)
    c2 = cflat.reshape(n, n)
    ct2 = ctflat.reshape(n, n)
    lap = _lap(c2, n)
    _, u = jnp.linalg.eigh(lap)
    h0 = u.T @ x
    h3 = _layers(ct2, h0, W1, b1, W2, b2, W3, b3)
    h4 = u @ h3
    return _post_pool(h4, batch.reshape(1, -1))


# speedup vs baseline: 1.0257x; 1.0011x over previous
import functools

import jax
import jax.numpy as jnp
from jax import lax
from jax.experimental import pallas as pl
from jax.experimental.pallas import tpu as pltpu
from jax.experimental.pallas import tpu_sc as plsc

_NC = 2
_NS = 16
_L = 16
_B = 16


def _build_counts(edge_flat, n, e):
    nw = _NC * _NS
    rch = 16
    nchunks = n // rch
    cpw = nchunks // nw
    eb = min(e, 16384)
    stages = e // eb
    vecs = eb // _L

    mesh = plsc.VectorSubcoreMesh(core_axis_name="c", subcore_axis_name="s")

    @functools.partial(
        pl.kernel,
        out_type=[
            jax.ShapeDtypeStruct((n * n,), jnp.float32),
            jax.ShapeDtypeStruct((n * n,), jnp.float32),
        ],
        mesh=mesh,
        compiler_params=pltpu.CompilerParams(needs_layout_passes=False),
        scratch_types=[
            pltpu.VMEM((eb,), jnp.int32),
            pltpu.VMEM((eb,), jnp.int32),
            pltpu.VMEM((rch * n,), jnp.float32),
            pltpu.VMEM((rch * n,), jnp.float32),
        ],
    )
    def scatter_kernel(edges_hbm, c_hbm, ct_hbm, rows_v, cols_v, accc_v, acct_v):
        wid = lax.axis_index("s") * _NC + lax.axis_index("c")
        ones = jnp.ones((_L,), jnp.float32)
        zeros = jnp.zeros((_L,), jnp.float32)

        def per_chunk(k_, carry):
            chunk = wid * cpw + k_
            r0 = chunk * rch

            def zero_body(i, c):
                accc_v[pl.ds(i * _L, _L)] = zeros
                acct_v[pl.ds(i * _L, _L)] = zeros
                return c

            lax.fori_loop(0, rch * n // _L, zero_body, 0)

            def stage_body(s_, c):
                pltpu.sync_copy(edges_hbm.at[pl.ds(s_ * eb, eb)], rows_v)
                pltpu.sync_copy(edges_hbm.at[pl.ds(e + s_ * eb, eb)], cols_v)

                def vec_body(j, c2):
                    r16 = rows_v[pl.ds(j * _L, _L)]
                    c16 = cols_v[pl.ds(j * _L, _L)]
                    mc = (r16 >= r0) & (r16 < r0 + rch)
                    offc = jnp.where(mc, (r16 - r0) * n + c16, 0)
                    plsc.addupdate_scatter(accc_v, [offc], ones, mask=mc)
                    mt = (c16 >= r0) & (c16 < r0 + rch)
                    offt = jnp.where(mt, (c16 - r0) * n + r16, 0)
                    plsc.addupdate_scatter(acct_v, [offt], ones, mask=mt)
                    return c2

                lax.fori_loop(0, vecs, vec_body, 0)
                return c

            lax.fori_loop(0, stages, stage_body, 0)

            pltpu.sync_copy(accc_v, c_hbm.at[pl.ds(chunk * rch * n, rch * n)])
            pltpu.sync_copy(acct_v, ct_hbm.at[pl.ds(chunk * rch * n, rch * n)])
            return carry

        lax.fori_loop(0, cpw, per_chunk, 0)

    return scatter_kernel(edge_flat)


def _lap(c2, n):
    bm = 256

    def body(c_ref, lap_ref):
        c = c_ref[...]
        adj = jnp.minimum(c, 1.0)
        deg = jnp.sum(adj, axis=1)
        i = pl.program_id(0)
        rows = lax.broadcasted_iota(jnp.int32, (bm, n), 0) + i * bm
        cols = lax.broadcasted_iota(jnp.int32, (bm, n), 1)
        lap_ref[...] = jnp.where(rows == cols, deg[:, None], 0.0) - adj

    return pl.pallas_call(
        body,
        grid=(n // bm,),
        in_specs=[pl.BlockSpec((bm, n), lambda i: (i, 0))],
        out_specs=pl.BlockSpec((bm, n), lambda i: (i, 0)),
        out_shape=jax.ShapeDtypeStruct((n, n), jnp.float32),
    )(c2)


def _post_pool(h4, batch2):
    n, d = h4.shape

    def body(h_ref, bt_ref, o_ref):
        bid = lax.broadcasted_iota(jnp.int32, (_B, n), 0)
        m = (bt_ref[...] == bid).astype(jnp.float32)
        pooled = jnp.dot(m, h_ref[...], preferred_element_type=jnp.float32)
        counts = jnp.sum(m, axis=1)[:, None]
        o_ref[...] = pooled / jnp.maximum(counts, 1.0)

    return pl.pallas_call(
        body,
        out_shape=jax.ShapeDtypeStruct((_B, d), jnp.float32),
    )(h4, batch2)


def _layers(ct2, h0, w1, b1, w2, b2, w3, b3):
    n = ct2.shape[0]
    dout = w3.shape[0]

    def body(ct_ref, h_ref, w1_ref, b1_ref, w2_ref, b2_ref, w3_ref, b3_ref,
             o_ref):
        ct = ct_ref[...]
        deg = jnp.sum(ct, axis=1) + 1.0
        dinv = lax.rsqrt(deg)[:, None]

        def layer(h, w, b, relu):
            xw = lax.dot_general(h, w, (((1,), (1,)), ((), ())),
                                 preferred_element_type=jnp.float32)
            y = dinv * xw
            s = jnp.dot(ct, y, preferred_element_type=jnp.float32) + y
            r = dinv * s + b
            return jnp.maximum(r, 0.0) if relu else r

        h = layer(h_ref[...], w1_ref[...], b1_ref[...], True)
        h = layer(h, w2_ref[...], b2_ref[...], True)
        o_ref[...] = layer(h, w3_ref[...], b3_ref[...], False)

    return pl.pallas_call(
        body,
        out_shape=jax.ShapeDtypeStruct((n, dout), jnp.float32),
    )(ct2, h0, w1, b1.reshape(1, -1), w2, b2.reshape(1, -1),
      w3, b3.reshape(1, -1))


def kernel(x, edge_index, batch, W1, b1, W2, b2, W3, b3):
    n = x.shape[0]
    e = edge_index.shape[1]
    cflat, ctflat = _build_counts(edge_index.reshape(-1), n, e)
    c2 = cflat.reshape(n, n)
    ct2 = ctflat.reshape(n, n)
    lap = _lap(c2, n)
    _, u = jnp.linalg.eigh(lap)
    h0 = u.T @ x
    h3 = _layers(ct2, h0, W1, b1, W2, b2, W3, b3)
    h4 = u @ h3
    return _post_pool(h4, batch.reshape(1, -1))
